# baseline (speedup 1.0000x reference)
import jax
import jax.numpy as jnp
from jax import lax
from jax.experimental import pallas as pl

R = 128
C = 32768
RB = 8
NB = R // RB

_BIG = 2**30


def _body(x_ref, o_ref):
    x = x_ref[...]
    m = jnp.max(x, axis=1, keepdims=True)
    col = lax.broadcasted_iota(jnp.int32, x.shape, 1)
    idx = jnp.min(jnp.where(x == m, col, _BIG), axis=1, keepdims=True)
    o_ref[...] = (col == idx).astype(jnp.float32)


def kernel(input):
    return pl.pallas_call(
        _body,
        grid=(NB,),
        in_specs=[pl.BlockSpec((RB, C), lambda b: (b, 0))],
        out_specs=pl.BlockSpec((RB, C), lambda b: (b, 0)),
        out_shape=jax.ShapeDtypeStruct((R, C), jnp.float32),
    )(input)


# speedup vs baseline: 1.5760x; 1.5760x over previous
import jax
import jax.numpy as jnp
from jax import lax
from jax.experimental import pallas as pl
from jax.experimental.pallas import tpu as pltpu

R = 128
C = 32768
BC = 8192
NB = C // BC

_BIG = 2**30


def _body(x_ref, o_ref, m_ref, i_ref):
    p = pl.program_id(0)
    b = pl.program_id(1)

    @pl.when(p == 0)
    def _pass0():
        x = x_ref[...]
        bm = jnp.max(x, axis=1, keepdims=True)
        col = lax.broadcasted_iota(jnp.int32, x.shape, 1) + b * BC
        bi = jnp.min(jnp.where(x == bm, col, _BIG), axis=1, keepdims=True)

        @pl.when(b == 0)
        def _():
            m_ref[...] = bm
            i_ref[...] = bi

        @pl.when(b != 0)
        def _():
            better = bm > m_ref[...]
            m_ref[...] = jnp.where(better, bm, m_ref[...])
            i_ref[...] = jnp.where(better, bi, i_ref[...])

    @pl.when(p == 1)
    def _pass1():
        col = lax.broadcasted_iota(jnp.int32, o_ref.shape, 1) + b * BC
        o_ref[...] = (col == i_ref[...]).astype(jnp.float32)


def kernel(input):
    return pl.pallas_call(
        _body,
        grid=(2, NB),
        in_specs=[
            pl.BlockSpec((R, BC), lambda p, b: (0, jnp.where(p == 0, b, NB - 1))),
        ],
        out_specs=pl.BlockSpec((R, BC), lambda p, b: (0, jnp.where(p == 0, 0, b))),
        out_shape=jax.ShapeDtypeStruct((R, C), jnp.float32),
        scratch_shapes=[
            pltpu.VMEM((R, 1), jnp.float32),
            pltpu.VMEM((R, 1), jnp.int32),
        ],
    )(input)
